# baseline (speedup 1.0000x reference)
import functools

import jax
import jax.numpy as jnp
from jax import lax
from jax.experimental import pallas as pl
from jax.experimental.pallas import tpu as pltpu
from jax.experimental.pallas import tpu_sc as plsc

N = 10000
NPOOL = 8000
NMISS = N - NPOOL
D = 128

RPAD = 2048
W = 10240
COLOFF = 2048
TOTAL = RPAD * W

E = 160000
EPT = 5008
EPAD = EPT * 32
NCHUNK = EPT // 16
NIDX = 2 * EPT
NROW = (NIDX + 127) // 128

_mesh = plsc.VectorSubcoreMesh(
    core_axis_name="c", subcore_axis_name="s", num_cores=2, num_subcores=16
)


def _sc_scatter_body(src_hbm, dst_hbm, p_hbm, srcv, dstv, idxb, onesb, sem):
    c = lax.axis_index("c")
    s = lax.axis_index("s")
    wid = c * 16 + s

    ebase = wid * EPT
    pltpu.sync_copy(src_hbm.at[pl.ds(ebase, EPT)], srcv)
    pltpu.sync_copy(dst_hbm.at[pl.ds(ebase, EPT)], dstv)

    for t in range(8):
        onesb[pl.ds(t * 16, 16)] = jnp.ones((16,), jnp.float32)

    dump = (2000 + wid) * W + 10048 + lax.iota(jnp.int32, 16)

    def _pf(i, carry):
        idxb[i // 8, 0, pl.ds((i % 8) * 16, 16)] = dump
        return carry
    lax.fori_loop(0, NROW * 8, _pf, 0)

    def _body(i, carry):
        sv = srcv[pl.ds(i * 16, 16)]
        dv = dstv[pl.ds(i * 16, 16)]
        cd = jnp.where(dv >= NPOOL, dv - NPOOL, dv + COLOFF)
        cs = jnp.where(sv >= NPOOL, sv - NPOOL, sv + COLOFF)
        idx1 = jnp.where(sv >= NPOOL, (sv - NPOOL) * W + cd, dump)
        idx2 = jnp.where(dv >= NPOOL, (dv - NPOOL) * W + cs, dump)
        idxb[i // 8, 0, pl.ds((i % 8) * 16, 16)] = idx1
        j = i + NCHUNK
        idxb[j // 8, 0, pl.ds((j % 8) * 16, 16)] = idx2
        return carry


    GRP = 8
    for g in range(0, NROW, GRP):
        copies = [
            pltpu.async_copy(onesb, p_hbm.at[idxb.at[j, 0]], sem)
            for j in range(g, min(g + GRP, NROW))
        ]
        for cp in copies:
            cp.wait()


_sc_scatter = functools.partial(
    pl.kernel,
    out_type=(),
    mesh=_mesh,
    scratch_types=[
        pltpu.VMEM((EPT,), jnp.int32),
        pltpu.VMEM((EPT,), jnp.int32),
        pltpu.VMEM((NROW, 1, 128), jnp.int32),
        pltpu.VMEM((128,), jnp.float32),
        pltpu.SemaphoreType.DMA,
    ],
)(_sc_scatter_body)


def _tc_solve_body(p_ref, x_ref, out_ref, val_ref):
    k = pl.program_id(0)

    @pl.when(k == 0)
    def _():
        val_ref[...] = jnp.zeros_like(val_ref)

    pblk = p_ref[...]
    base = jnp.dot(pblk, x_ref[...], preferred_element_type=jnp.float32)

    mrow = pblk[:, :RPAD]
    mkk = p_ref[:, pl.ds(k * 128, 128)]
    rr = lax.broadcasted_iota(jnp.int32, (128, 128), 0)
    cc = lax.broadcasted_iota(jnp.int32, (128, 128), 1)
    diag = jnp.sum(jnp.where(rr == cc, mkk, 0.0), axis=1)
    wcols = lax.broadcasted_iota(jnp.int32, (1, W), 1)
    cnt = jnp.sum(pblk * (wcols < COLOFF + NPOOL).astype(jnp.float32), axis=1) - diag

    r = base + jnp.dot(mrow, val_ref[...], preferred_element_type=jnp.float32)

    dinv = (1.0 / jnp.maximum(cnt, 1.0))[:, None]
    a = jnp.where(rr > cc, mkk, 0.0) * dinv
    y = r * dinv
    t = a
    for _ in range(6):
        y = y + jnp.dot(t, y, preferred_element_type=jnp.float32)
        t = jnp.dot(t, t, preferred_element_type=jnp.float32)
    y = y + jnp.dot(t, y, preferred_element_type=jnp.float32)

    val_ref[pl.ds(k * 128, 128), :] = y
    out_ref[...] = y


def _tc_solve(p2d, xfull):
    return pl.pallas_call(
        _tc_solve_body,
        grid=(RPAD // 128,),
        in_specs=[
            pl.BlockSpec((128, W), lambda k: (k, 0)),
            pl.BlockSpec((W, D), lambda k: (0, 0)),
        ],
        out_specs=pl.BlockSpec((128, D), lambda k: (k, 0)),
        out_shape=jax.ShapeDtypeStruct((RPAD, D), jnp.float32),
        scratch_shapes=[pltpu.VMEM((RPAD, D), jnp.float32)],
    )(p2d, xfull)


def kernel(x_abstract, perm, edge_index, original_num_nodes):
    src = jnp.concatenate(
        [edge_index[0], jnp.zeros((EPAD - E,), jnp.int32)]
    )
    dst = jnp.concatenate(
        [edge_index[1], jnp.zeros((EPAD - E,), jnp.int32)]
    )
    p_state = jax.new_ref(jnp.zeros((TOTAL,), jnp.float32))
    _sc_scatter(src, dst, p_state)
    p2d = p_state[...].reshape(RPAD, W)
    xfull = jnp.concatenate(
        [
            jnp.zeros((COLOFF, D), jnp.float32),
            x_abstract,
            jnp.zeros((W - COLOFF - NPOOL, D), jnp.float32),
        ]
    )
    val = _tc_solve(p2d, xfull)
    return jnp.concatenate([x_abstract, val[:NMISS]], axis=0)


# speedup vs baseline: 15.4119x; 15.4119x over previous
import functools

import jax
import jax.numpy as jnp
from jax import lax
from jax.experimental import pallas as pl
from jax.experimental.pallas import tpu as pltpu
from jax.experimental.pallas import tpu_sc as plsc

N = 10000
NPOOL = 8000
NMISS = N - NPOOL
D = 128

RPAD = 2048
W = 10240
COLOFF = 2048
TOTAL = RPAD * W

E = 160000
EPT = 5008
EPAD = EPT * 32
NCHUNK = EPT // 16
NIDX = 2 * EPT
NROW = (NIDX + 127) // 128

_mesh = plsc.VectorSubcoreMesh(
    core_axis_name="c", subcore_axis_name="s", num_cores=2, num_subcores=16
)


def _sc_scatter_body(src_hbm, dst_hbm, p_hbm, srcv, dstv, idx1d, idxb, onesb, sem):
    c = lax.axis_index("c")
    s = lax.axis_index("s")
    wid = c * 16 + s

    ebase = wid * EPT
    pltpu.sync_copy(src_hbm.at[pl.ds(ebase, EPT)], srcv)
    pltpu.sync_copy(dst_hbm.at[pl.ds(ebase, EPT)], dstv)

    for t in range(8):
        onesb[pl.ds(t * 16, 16)] = jnp.ones((16,), jnp.float32)

    def _body(i, ptr):
        sv = srcv[pl.ds(i * 16, 16)]
        dv = dstv[pl.ds(i * 16, 16)]
        cd = jnp.where(dv >= NPOOL, dv - NPOOL, dv + COLOFF)
        cs = jnp.where(sv >= NPOOL, sv - NPOOL, sv + COLOFF)
        m1 = sv >= NPOOL
        m2 = dv >= NPOOL
        pos1 = ptr + plsc.cumsum(m1.astype(jnp.int32)) - 1
        plsc.store_scatter(idx1d, [pos1], (sv - NPOOL) * W + cd, mask=m1)
        ptr = ptr + plsc.all_reduce_population_count(m1)[0]
        pos2 = ptr + plsc.cumsum(m2.astype(jnp.int32)) - 1
        plsc.store_scatter(idx1d, [pos2], (dv - NPOOL) * W + cs, mask=m2)
        return ptr + plsc.all_reduce_population_count(m2)[0]

    ptr = lax.fori_loop(0, NCHUNK, _body, 0)

    for t in range(8):
        idx1d[pl.ds(ptr + t * 16, 16)] = (
            (2000 + wid) * W + 10048 + t * 16 + lax.iota(jnp.int32, 16)
        )

    nch = (ptr + 127) // 128

    def _repack(q, carry):
        idxb[q // 8, 0, pl.ds((q % 8) * 16, 16)] = idx1d[pl.ds(q * 16, 16)]
        return carry

    lax.fori_loop(0, nch * 8, _repack, 0)

    def _fire(j, carry):
        pltpu.async_copy(onesb, p_hbm.at[idxb.at[j, 0]], sem).wait()
        return carry

    lax.fori_loop(0, nch, _fire, 0)


_sc_scatter = functools.partial(
    pl.kernel,
    out_type=(),
    mesh=_mesh,
    scratch_types=[
        pltpu.VMEM((EPT,), jnp.int32),
        pltpu.VMEM((EPT,), jnp.int32),
        pltpu.VMEM((NROW * 128 + 144,), jnp.int32),
        pltpu.VMEM((NROW, 1, 128), jnp.int32),
        pltpu.VMEM((128,), jnp.float32),
        pltpu.SemaphoreType.DMA,
    ],
    compiler_params=pltpu.CompilerParams(needs_layout_passes=False),
)(_sc_scatter_body)


def _tc_solve_body(p_ref, x_ref, out_ref, val_ref):
    k = pl.program_id(0)

    @pl.when(k == 0)
    def _():
        val_ref[...] = jnp.zeros_like(val_ref)

    pblk = p_ref[...]
    base = jnp.dot(pblk, x_ref[...], preferred_element_type=jnp.float32)

    mrow = pblk[:, :RPAD]
    mkk = p_ref[:, pl.ds(k * 128, 128)]
    rr = lax.broadcasted_iota(jnp.int32, (128, 128), 0)
    cc = lax.broadcasted_iota(jnp.int32, (128, 128), 1)
    diag = jnp.sum(jnp.where(rr == cc, mkk, 0.0), axis=1)
    wcols = lax.broadcasted_iota(jnp.int32, (1, W), 1)
    cnt = jnp.sum(pblk * (wcols < COLOFF + NPOOL).astype(jnp.float32), axis=1) - diag

    r = base + jnp.dot(mrow, val_ref[...], preferred_element_type=jnp.float32)

    dinv = (1.0 / jnp.maximum(cnt, 1.0))[:, None]
    a = jnp.where(rr > cc, mkk, 0.0) * dinv
    y = r * dinv
    t = a
    for _ in range(6):
        y = y + jnp.dot(t, y, preferred_element_type=jnp.float32)
        t = jnp.dot(t, t, preferred_element_type=jnp.float32)
    y = y + jnp.dot(t, y, preferred_element_type=jnp.float32)

    val_ref[pl.ds(k * 128, 128), :] = y
    out_ref[...] = y


def _tc_solve(p2d, xfull):
    return pl.pallas_call(
        _tc_solve_body,
        grid=(RPAD // 128,),
        in_specs=[
            pl.BlockSpec((128, W), lambda k: (k, 0)),
            pl.BlockSpec((W, D), lambda k: (0, 0)),
        ],
        out_specs=pl.BlockSpec((128, D), lambda k: (k, 0)),
        out_shape=jax.ShapeDtypeStruct((RPAD, D), jnp.float32),
        scratch_shapes=[pltpu.VMEM((RPAD, D), jnp.float32)],
    )(p2d, xfull)


def kernel(x_abstract, perm, edge_index, original_num_nodes):
    src = jnp.concatenate(
        [edge_index[0], jnp.zeros((EPAD - E,), jnp.int32)]
    )
    dst = jnp.concatenate(
        [edge_index[1], jnp.zeros((EPAD - E,), jnp.int32)]
    )
    p_state = jax.new_ref(jnp.zeros((TOTAL,), jnp.float32))
    _sc_scatter(src, dst, p_state)
    p2d = p_state[...].reshape(RPAD, W)
    xfull = jnp.concatenate(
        [
            jnp.zeros((COLOFF, D), jnp.float32),
            x_abstract,
            jnp.zeros((W - COLOFF - NPOOL, D), jnp.float32),
        ]
    )
    val = _tc_solve(p2d, xfull)
    return jnp.concatenate([x_abstract, val[:NMISS]], axis=0)
